# baseline (speedup 1.0000x reference)
import jax
import jax.numpy as jnp
from jax.experimental import pallas as pl

_W = 4
_B = 512


def _body(x_ref, o_ref):
    o_ref[...] = x_ref[:, : o_ref.shape[1]]


def kernel(x):
    b, s, d = x.shape
    h = s // _W
    n = b * h
    xv = x.reshape(n, _W * d)
    out = pl.pallas_call(
        _body,
        grid=(n // _B,),
        in_specs=[pl.BlockSpec((_B, _W * d), lambda i: (i, 0))],
        out_specs=pl.BlockSpec((_B, d), lambda i: (i, 0)),
        out_shape=jax.ShapeDtypeStruct((n, d), x.dtype),
    )(xv)
    return out.reshape(b, h, d)
